# baseline (speedup 1.0000x reference)
import jax
import jax.numpy as jnp
from jax import lax
from jax.experimental import pallas as pl
from jax.experimental.pallas import tpu as pltpu
from jax.experimental.pallas import tpu_sc as plsc

N_USERS = 50000
N_ITEMS = 50000
N = N_USERS + N_ITEMS
D = 64
N_LAYERS = 3

N_PAD = 100352
TRASH = N
E = 2 * 500000 + N
E_PAD = 1179648
IW = 128
E_ROWS = E_PAD // IW
CHUNK = 6

NSUB = 16
ROWS_PER_SUB = N_PAD // NSUB
SUB_EROWS = E_ROWS // NSUB
SUB_CHUNKS = SUB_EROWS // CHUNK
NZCH = ROWS_PER_SUB // 128


def _zero_vmem(ref, nrows):
    zeros = jnp.zeros((16,), jnp.float32)

    def body(i, carry):
        ref[i, :] = zeros
        return carry

    lax.fori_loop(0, nrows, body, 0)


def _spmm_body(g4_hbm, row_hbm, col_hbm, s4_hbm, idxr_a, idxc_a, idxr_b,
               idxc_b, rows_a, rows_b, zb, sh, gsem_a, gsem_b, ssem_a,
               ssem_b):
    c = lax.axis_index("c")
    s = lax.axis_index("s")
    _zero_vmem(zb, 128)

    slots = ((idxr_a, idxc_a, rows_a, gsem_a, ssem_a),
             (idxr_b, idxc_b, rows_b, gsem_b, ssem_b))

    for blk_i in range(2):
        blk = 2 * c + blk_i
        g_blk = g4_hbm.at[blk]
        out_blk = s4_hbm.at[blk]

        def load_fire(chunk, slot):
            idxr, idxc, rows, gsem, _ = slots[slot]
            base = s * SUB_EROWS + chunk * CHUNK
            pltpu.sync_copy(row_hbm.at[pl.ds(base, CHUNK)], idxr)
            pltpu.sync_copy(col_hbm.at[pl.ds(base, CHUNK)], idxc)
            for j in range(CHUNK):
                pltpu.async_copy(g_blk.at[idxr.at[j]], rows.at[j], gsem)

        def wait_g(slot):
            idxr, _, rows, gsem, _ = slots[slot]
            for j in range(CHUNK):
                pltpu.make_async_copy(g_blk.at[idxr.at[j]], rows.at[j],
                                      gsem).wait()

        def fire_s(slot):
            _, idxc, rows, _, ssem = slots[slot]
            for j in range(CHUNK):
                pltpu.async_copy(rows.at[j], sh.at[idxc.at[j]], ssem,
                                 add=True)

        def wait_s(slot):
            _, idxc, rows, _, ssem = slots[slot]
            for j in range(CHUNK):
                pltpu.make_async_copy(rows.at[j], sh.at[idxc.at[j]],
                                      ssem).wait()

        def zero_body(i, carry):
            pltpu.sync_copy(zb, sh.at[pl.ds(s * ROWS_PER_SUB + i * 128, 128)])
            return carry

        lax.fori_loop(0, NZCH, zero_body, 0)
        plsc.subcore_barrier()

        def pair_body(i, carry):
            @pl.when(i > 0)
            def _():
                wait_s(0)
            load_fire(2 * i, 0)

            @pl.when(i > 0)
            def _():
                wait_g(1)
                fire_s(1)
                wait_s(1)
            load_fire(2 * i + 1, 1)
            wait_g(0)
            fire_s(0)
            return carry

        lax.fori_loop(0, SUB_CHUNKS // 2, pair_body, 0)
        wait_g(1)
        fire_s(1)
        wait_s(0)
        wait_s(1)
        plsc.subcore_barrier()

        pltpu.sync_copy(sh.at[pl.ds(s * ROWS_PER_SUB, ROWS_PER_SUB)],
                        out_blk.at[pl.ds(s * ROWS_PER_SUB, ROWS_PER_SUB)])
        plsc.subcore_barrier()


def _hist_body(col_hbm, out_hbm, idxc, ones_v, zb1, sh, sem):
    c = lax.axis_index("c")
    s = lax.axis_index("s")

    def zfill_body(i, carry):
        zb1[pl.ds(i * 16, 16)] = jnp.zeros((16,), jnp.float32)
        return carry

    lax.fori_loop(0, 2048 // 16, zfill_body, 0)

    def ones_body(i, carry):
        ones_v[pl.ds(i * 16, 16)] = jnp.ones((16,), jnp.float32)
        return carry

    lax.fori_loop(0, IW // 16, ones_body, 0)

    def zero_body(i, carry):
        pltpu.sync_copy(zb1, sh.at[pl.ds(s * ROWS_PER_SUB + i * 2048, 2048)])
        return carry

    nz = ROWS_PER_SUB // 2048
    lax.fori_loop(0, nz, zero_body, 0)
    rem = ROWS_PER_SUB - nz * 2048
    pltpu.sync_copy(zb1.at[pl.ds(0, rem)],
                    sh.at[pl.ds(s * ROWS_PER_SUB + nz * 2048, rem)])
    plsc.subcore_barrier()

    half_rows = E_ROWS // 2
    sub_rows = half_rows // NSUB
    nchunks = sub_rows // CHUNK

    def chunk_body(t, carry):
        base = c * half_rows + s * sub_rows + t * CHUNK
        pltpu.sync_copy(col_hbm.at[pl.ds(base, CHUNK)], idxc)
        adds = [
            pltpu.async_copy(ones_v, sh.at[idxc.at[j]], sem, add=True)
            for j in range(CHUNK)
        ]
        for d in adds:
            d.wait()
        return carry

    lax.fori_loop(0, nchunks, chunk_body, 0)
    plsc.subcore_barrier()

    pltpu.sync_copy(sh.at[pl.ds(s * ROWS_PER_SUB, ROWS_PER_SUB)],
                    out_hbm.at[c].at[pl.ds(s * ROWS_PER_SUB, ROWS_PER_SUB)])


_MESH = plsc.VectorSubcoreMesh(core_axis_name="c", subcore_axis_name="s")

_spmm = pl.kernel(
    _spmm_body,
    out_type=jax.ShapeDtypeStruct((4, N_PAD, 16), jnp.float32),
    mesh=_MESH,
    compiler_params=pltpu.CompilerParams(use_tc_tiling_on_sc=False),
    scratch_types=[
        pltpu.VMEM((CHUNK, IW), jnp.int32),
        pltpu.VMEM((CHUNK, IW), jnp.int32),
        pltpu.VMEM((CHUNK, IW), jnp.int32),
        pltpu.VMEM((CHUNK, IW), jnp.int32),
        pltpu.VMEM((CHUNK, IW, 16), jnp.float32),
        pltpu.VMEM((CHUNK, IW, 16), jnp.float32),
        pltpu.VMEM((128, 16), jnp.float32),
        pltpu.VMEM_SHARED((N_PAD, 16), jnp.float32),
        pltpu.SemaphoreType.DMA,
        pltpu.SemaphoreType.DMA,
        pltpu.SemaphoreType.DMA,
        pltpu.SemaphoreType.DMA,
    ],
)

_hist = pl.kernel(
    _hist_body,
    out_type=jax.ShapeDtypeStruct((2, N_PAD), jnp.float32),
    mesh=_MESH,
    compiler_params=pltpu.CompilerParams(use_tc_tiling_on_sc=False),
    scratch_types=[
        pltpu.VMEM((CHUNK, IW), jnp.int32),
        pltpu.VMEM((IW,), jnp.float32),
        pltpu.VMEM((2048,), jnp.float32),
        pltpu.VMEM_SHARED((N_PAD,), jnp.float32),
        pltpu.SemaphoreType.DMA,
    ],
)


@jax.jit
def kernel(edge_index, user_w, item_w, aspect_weight):
    edge_index = edge_index.astype(jnp.int32)
    src = edge_index[0]
    dst = edge_index[1] + N_USERS
    loops = jnp.arange(N, dtype=jnp.int32)
    row = jnp.concatenate([src, dst, loops])
    col = jnp.concatenate([dst, src, loops])
    pad = E_PAD - E
    row2d = jnp.concatenate([row, jnp.zeros((pad,), jnp.int32)]).reshape(E_ROWS, IW)
    col2d = jnp.concatenate([col, jnp.full((pad,), TRASH, jnp.int32)]).reshape(E_ROWS, IW)

    hist = _hist(col2d)
    deg = hist[0] + hist[1]
    deg = jnp.maximum(deg, 1.0)
    d2f = jnp.broadcast_to((1.0 / deg)[:, None], (N_PAD, 16)).reshape(1, -1)
    dinvf = jnp.broadcast_to(jax.lax.rsqrt(deg)[:, None], (N_PAD, 16)).reshape(1, -1)
    dsqf = 0.25 / dinvf

    x = jnp.concatenate([user_w, item_w], axis=0)
    x = jnp.pad(x, ((0, N_PAD - N), (0, 0)))
    x4f = jnp.transpose(x.reshape(N_PAD, 4, 16), (1, 0, 2)).reshape(4, -1)

    g = x4f * dinvf
    G = g
    for _ in range(N_LAYERS):
        S = _spmm(g.reshape(4, N_PAD, 16), row2d, col2d).reshape(4, -1)
        g = S * d2f
        G = G + g

    interest4 = (G * dsqf).reshape(4, N_PAD, 16)
    interest = jnp.transpose(interest4, (1, 0, 2)).reshape(N_PAD, D)[:N]

    alpha = jax.nn.softmax(aspect_weight, axis=0)
    user_final = alpha[0] * interest[:N_USERS] + alpha[1] * user_w
    item_final = interest[N_USERS:]
    return user_final, item_final


# speedup vs baseline: 1.0047x; 1.0047x over previous
import jax
import jax.numpy as jnp
from jax import lax
from jax.experimental import pallas as pl
from jax.experimental.pallas import tpu as pltpu
from jax.experimental.pallas import tpu_sc as plsc

N_USERS = 50000
N_ITEMS = 50000
N = N_USERS + N_ITEMS
D = 64
N_LAYERS = 3

N_PAD = 100352
TRASH = N
E = 2 * 500000 + N
E_PAD = 1179648
IW = 128
E_ROWS = E_PAD // IW
CHUNK = 4

NSUB = 16
ROWS_PER_SUB = N_PAD // NSUB
SUB_EROWS = E_ROWS // NSUB
SUB_CHUNKS = SUB_EROWS // CHUNK
NZCH = ROWS_PER_SUB // 128


def _zero_vmem(ref, nrows):
    zeros = jnp.zeros((16,), jnp.float32)

    def body(i, carry):
        ref[i, :] = zeros
        return carry

    lax.fori_loop(0, nrows, body, 0)


def _spmm_body(g4_hbm, row_hbm, col_hbm, s4_hbm, idxr_a, idxc_a, idxr_b,
               idxc_b, rows_a, rows_b, zb, sh, gsem_a, gsem_b, ssem_a,
               ssem_b):
    c = lax.axis_index("c")
    s = lax.axis_index("s")
    _zero_vmem(zb, 128)

    slots = ((idxr_a, idxc_a, rows_a, gsem_a, ssem_a),
             (idxr_b, idxc_b, rows_b, gsem_b, ssem_b))

    for blk_i in range(2):
        blk = 2 * c + blk_i
        g_blk = g4_hbm.at[blk]
        out_blk = s4_hbm.at[blk]

        def load_fire(chunk, slot):
            idxr, idxc, rows, gsem, _ = slots[slot]
            base = s * SUB_EROWS + chunk * CHUNK
            pltpu.sync_copy(row_hbm.at[pl.ds(base, CHUNK)], idxr)
            pltpu.sync_copy(col_hbm.at[pl.ds(base, CHUNK)], idxc)
            for j in range(CHUNK):
                pltpu.async_copy(g_blk.at[idxr.at[j]], rows.at[j], gsem)

        def wait_g(slot):
            idxr, _, rows, gsem, _ = slots[slot]
            for j in range(CHUNK):
                pltpu.make_async_copy(g_blk.at[idxr.at[j]], rows.at[j],
                                      gsem).wait()

        def fire_s(slot):
            _, idxc, rows, _, ssem = slots[slot]
            for j in range(CHUNK):
                pltpu.async_copy(rows.at[j], sh.at[idxc.at[j]], ssem,
                                 add=True)

        def wait_s(slot):
            _, idxc, rows, _, ssem = slots[slot]
            for j in range(CHUNK):
                pltpu.make_async_copy(rows.at[j], sh.at[idxc.at[j]],
                                      ssem).wait()

        def zero_body(i, carry):
            pltpu.sync_copy(zb, sh.at[pl.ds(s * ROWS_PER_SUB + i * 128, 128)])
            return carry

        lax.fori_loop(0, NZCH, zero_body, 0)
        plsc.subcore_barrier()

        def pair_body(i, carry):
            @pl.when(i > 0)
            def _():
                wait_s(0)
            load_fire(2 * i, 0)

            @pl.when(i > 0)
            def _():
                wait_g(1)
                fire_s(1)
                wait_s(1)
            load_fire(2 * i + 1, 1)
            wait_g(0)
            fire_s(0)
            return carry

        lax.fori_loop(0, SUB_CHUNKS // 2, pair_body, 0)
        wait_g(1)
        fire_s(1)
        wait_s(0)
        wait_s(1)
        plsc.subcore_barrier()

        pltpu.sync_copy(sh.at[pl.ds(s * ROWS_PER_SUB, ROWS_PER_SUB)],
                        out_blk.at[pl.ds(s * ROWS_PER_SUB, ROWS_PER_SUB)])
        plsc.subcore_barrier()


def _hist_body(col_hbm, out_hbm, idxc, ones_v, zb1, sh, sem):
    c = lax.axis_index("c")
    s = lax.axis_index("s")

    def zfill_body(i, carry):
        zb1[pl.ds(i * 16, 16)] = jnp.zeros((16,), jnp.float32)
        return carry

    lax.fori_loop(0, 2048 // 16, zfill_body, 0)

    def ones_body(i, carry):
        ones_v[pl.ds(i * 16, 16)] = jnp.ones((16,), jnp.float32)
        return carry

    lax.fori_loop(0, IW // 16, ones_body, 0)

    def zero_body(i, carry):
        pltpu.sync_copy(zb1, sh.at[pl.ds(s * ROWS_PER_SUB + i * 2048, 2048)])
        return carry

    nz = ROWS_PER_SUB // 2048
    lax.fori_loop(0, nz, zero_body, 0)
    rem = ROWS_PER_SUB - nz * 2048
    pltpu.sync_copy(zb1.at[pl.ds(0, rem)],
                    sh.at[pl.ds(s * ROWS_PER_SUB + nz * 2048, rem)])
    plsc.subcore_barrier()

    half_rows = E_ROWS // 2
    sub_rows = half_rows // NSUB
    nchunks = sub_rows // CHUNK

    def chunk_body(t, carry):
        base = c * half_rows + s * sub_rows + t * CHUNK
        pltpu.sync_copy(col_hbm.at[pl.ds(base, CHUNK)], idxc)
        adds = [
            pltpu.async_copy(ones_v, sh.at[idxc.at[j]], sem, add=True)
            for j in range(CHUNK)
        ]
        for d in adds:
            d.wait()
        return carry

    lax.fori_loop(0, nchunks, chunk_body, 0)
    plsc.subcore_barrier()

    pltpu.sync_copy(sh.at[pl.ds(s * ROWS_PER_SUB, ROWS_PER_SUB)],
                    out_hbm.at[c].at[pl.ds(s * ROWS_PER_SUB, ROWS_PER_SUB)])


_MESH = plsc.VectorSubcoreMesh(core_axis_name="c", subcore_axis_name="s")

_spmm = pl.kernel(
    _spmm_body,
    out_type=jax.ShapeDtypeStruct((4, N_PAD, 16), jnp.float32),
    mesh=_MESH,
    compiler_params=pltpu.CompilerParams(use_tc_tiling_on_sc=False),
    scratch_types=[
        pltpu.VMEM((CHUNK, IW), jnp.int32),
        pltpu.VMEM((CHUNK, IW), jnp.int32),
        pltpu.VMEM((CHUNK, IW), jnp.int32),
        pltpu.VMEM((CHUNK, IW), jnp.int32),
        pltpu.VMEM((CHUNK, IW, 16), jnp.float32),
        pltpu.VMEM((CHUNK, IW, 16), jnp.float32),
        pltpu.VMEM((128, 16), jnp.float32),
        pltpu.VMEM_SHARED((N_PAD, 16), jnp.float32),
        pltpu.SemaphoreType.DMA,
        pltpu.SemaphoreType.DMA,
        pltpu.SemaphoreType.DMA,
        pltpu.SemaphoreType.DMA,
    ],
)

_hist = pl.kernel(
    _hist_body,
    out_type=jax.ShapeDtypeStruct((2, N_PAD), jnp.float32),
    mesh=_MESH,
    compiler_params=pltpu.CompilerParams(use_tc_tiling_on_sc=False),
    scratch_types=[
        pltpu.VMEM((CHUNK, IW), jnp.int32),
        pltpu.VMEM((IW,), jnp.float32),
        pltpu.VMEM((2048,), jnp.float32),
        pltpu.VMEM_SHARED((N_PAD,), jnp.float32),
        pltpu.SemaphoreType.DMA,
    ],
)


@jax.jit
def kernel(edge_index, user_w, item_w, aspect_weight):
    edge_index = edge_index.astype(jnp.int32)
    src = edge_index[0]
    dst = edge_index[1] + N_USERS
    loops = jnp.arange(N, dtype=jnp.int32)
    row = jnp.concatenate([src, dst, loops])
    col = jnp.concatenate([dst, src, loops])
    pad = E_PAD - E
    row2d = jnp.concatenate([row, jnp.zeros((pad,), jnp.int32)]).reshape(E_ROWS, IW)
    col2d = jnp.concatenate([col, jnp.full((pad,), TRASH, jnp.int32)]).reshape(E_ROWS, IW)

    hist = _hist(col2d)
    deg = hist[0] + hist[1]
    deg = jnp.maximum(deg, 1.0)
    d2f = jnp.broadcast_to((1.0 / deg)[:, None], (N_PAD, 16)).reshape(1, -1)
    dinvf = jnp.broadcast_to(jax.lax.rsqrt(deg)[:, None], (N_PAD, 16)).reshape(1, -1)
    dsqf = 0.25 / dinvf

    x = jnp.concatenate([user_w, item_w], axis=0)
    x = jnp.pad(x, ((0, N_PAD - N), (0, 0)))
    x4f = jnp.transpose(x.reshape(N_PAD, 4, 16), (1, 0, 2)).reshape(4, -1)

    g = x4f * dinvf
    G = g
    for _ in range(N_LAYERS):
        S = _spmm(g.reshape(4, N_PAD, 16), row2d, col2d).reshape(4, -1)
        g = S * d2f
        G = G + g

    interest4 = (G * dsqf).reshape(4, N_PAD, 16)
    interest = jnp.transpose(interest4, (1, 0, 2)).reshape(N_PAD, D)[:N]

    alpha = jax.nn.softmax(aspect_weight, axis=0)
    user_final = alpha[0] * interest[:N_USERS] + alpha[1] * user_w
    item_final = interest[N_USERS:]
    return user_final, item_final


# speedup vs baseline: 1.6076x; 1.6001x over previous
import jax
import jax.numpy as jnp
from jax import lax
from jax.experimental import pallas as pl
from jax.experimental.pallas import tpu as pltpu
from jax.experimental.pallas import tpu_sc as plsc

N_USERS = 50000
N_ITEMS = 50000
N = N_USERS + N_ITEMS
D = 64
N_LAYERS = 3

N_PAD = 100352
TRASH = N
E = 2 * 500000 + N
E_PAD = 1114112
IW = 128
E_ROWS = E_PAD // IW
CHUNK = 4

NSUB = 16
ROWS_PER_SUB = N_PAD // NSUB
SUB_EROWS = E_ROWS // NSUB
SUB_CHUNKS = SUB_EROWS // CHUNK
NZCH = ROWS_PER_SUB // 128


def _zero_vmem(ref, nrows):
    zeros = jnp.zeros((16,), jnp.float32)

    def body(i, carry):
        ref[i, :] = zeros
        return carry

    lax.fori_loop(0, nrows, body, 0)


def _spmm_body(g4_hbm, row_hbm, col_hbm, s4_hbm, idxr_a, idxc_a, idxr_b,
               idxc_b, rows_a, rows_b, zb, sh, gsem_a, gsem_b, ssem_a,
               ssem_b):
    c = lax.axis_index("c")
    s = lax.axis_index("s")
    _zero_vmem(zb, 128)

    slots = ((idxr_a, idxc_a, rows_a, gsem_a, ssem_a),
             (idxr_b, idxc_b, rows_b, gsem_b, ssem_b))

    for blk_i in range(2):
        blk = 2 * c + blk_i
        g_blk = g4_hbm.at[blk]
        out_blk = s4_hbm.at[blk]

        def load_fire(chunk, slot):
            idxr, idxc, rows, gsem, _ = slots[slot]
            base = s * SUB_EROWS + chunk * CHUNK
            pltpu.sync_copy(row_hbm.at[pl.ds(base, CHUNK)], idxr)
            pltpu.sync_copy(col_hbm.at[pl.ds(base, CHUNK)], idxc)
            for j in range(CHUNK):
                pltpu.async_copy(g_blk.at[idxr.at[j]], rows.at[j], gsem)

        def wait_g(slot):
            idxr, _, rows, gsem, _ = slots[slot]
            for j in range(CHUNK):
                pltpu.make_async_copy(g_blk.at[idxr.at[j]], rows.at[j],
                                      gsem).wait()

        def fire_s(slot):
            _, idxc, rows, _, ssem = slots[slot]
            for j in range(CHUNK):
                pltpu.async_copy(rows.at[j], sh.at[idxc.at[j]], ssem,
                                 add=True)

        def wait_s(slot):
            _, idxc, rows, _, ssem = slots[slot]
            for j in range(CHUNK):
                pltpu.make_async_copy(rows.at[j], sh.at[idxc.at[j]],
                                      ssem).wait()

        def zero_body(i, carry):
            pltpu.sync_copy(zb, sh.at[pl.ds(s * ROWS_PER_SUB + i * 128, 128)])
            return carry

        lax.fori_loop(0, NZCH, zero_body, 0)
        plsc.subcore_barrier()

        def pair_body(i, carry):
            @pl.when(i > 0)
            def _():
                wait_s(0)
            load_fire(2 * i, 0)

            @pl.when(i > 0)
            def _():
                wait_g(1)
                fire_s(1)
                wait_s(1)
            load_fire(2 * i + 1, 1)
            wait_g(0)
            fire_s(0)
            return carry

        lax.fori_loop(0, SUB_CHUNKS // 2, pair_body, 0)
        wait_g(1)
        fire_s(1)
        wait_s(0)
        wait_s(1)
        plsc.subcore_barrier()

        pltpu.sync_copy(sh.at[pl.ds(s * ROWS_PER_SUB, ROWS_PER_SUB)],
                        out_blk.at[pl.ds(s * ROWS_PER_SUB, ROWS_PER_SUB)])
        plsc.subcore_barrier()


def _hist_body(col_hbm, out_hbm, idxc, ones_v, zb1, sh, sem):
    c = lax.axis_index("c")
    s = lax.axis_index("s")

    def zfill_body(i, carry):
        zb1[pl.ds(i * 16, 16)] = jnp.zeros((16,), jnp.float32)
        return carry

    lax.fori_loop(0, 2048 // 16, zfill_body, 0)

    def ones_body(i, carry):
        ones_v[pl.ds(i * 16, 16)] = jnp.ones((16,), jnp.float32)
        return carry

    lax.fori_loop(0, IW // 16, ones_body, 0)

    def zero_body(i, carry):
        pltpu.sync_copy(zb1, sh.at[pl.ds(s * ROWS_PER_SUB + i * 2048, 2048)])
        return carry

    nz = ROWS_PER_SUB // 2048
    lax.fori_loop(0, nz, zero_body, 0)
    rem = ROWS_PER_SUB - nz * 2048
    pltpu.sync_copy(zb1.at[pl.ds(0, rem)],
                    sh.at[pl.ds(s * ROWS_PER_SUB + nz * 2048, rem)])
    plsc.subcore_barrier()

    half_rows = E_ROWS // 2
    sub_rows = half_rows // NSUB
    nchunks = sub_rows // CHUNK

    def chunk_body(t, carry):
        base = c * half_rows + s * sub_rows + t * CHUNK
        pltpu.sync_copy(col_hbm.at[pl.ds(base, CHUNK)], idxc)
        adds = [
            pltpu.async_copy(ones_v, sh.at[idxc.at[j]], sem, add=True)
            for j in range(CHUNK)
        ]
        for d in adds:
            d.wait()
        return carry

    lax.fori_loop(0, nchunks, chunk_body, 0)
    plsc.subcore_barrier()

    pltpu.sync_copy(sh.at[pl.ds(s * ROWS_PER_SUB, ROWS_PER_SUB)],
                    out_hbm.at[c].at[pl.ds(s * ROWS_PER_SUB, ROWS_PER_SUB)])


_MESH = plsc.VectorSubcoreMesh(core_axis_name="c", subcore_axis_name="s")

_spmm = pl.kernel(
    _spmm_body,
    out_type=jax.ShapeDtypeStruct((4, N_PAD, 16), jnp.float32),
    mesh=_MESH,
    compiler_params=pltpu.CompilerParams(use_tc_tiling_on_sc=False),
    scratch_types=[
        pltpu.VMEM((CHUNK, IW), jnp.int32),
        pltpu.VMEM((CHUNK, IW), jnp.int32),
        pltpu.VMEM((CHUNK, IW), jnp.int32),
        pltpu.VMEM((CHUNK, IW), jnp.int32),
        pltpu.VMEM((CHUNK, IW, 16), jnp.float32),
        pltpu.VMEM((CHUNK, IW, 16), jnp.float32),
        pltpu.VMEM((128, 16), jnp.float32),
        pltpu.VMEM_SHARED((N_PAD, 16), jnp.float32),
        pltpu.SemaphoreType.DMA,
        pltpu.SemaphoreType.DMA,
        pltpu.SemaphoreType.DMA,
        pltpu.SemaphoreType.DMA,
    ],
)

_hist = pl.kernel(
    _hist_body,
    out_type=jax.ShapeDtypeStruct((2, N_PAD), jnp.float32),
    mesh=_MESH,
    compiler_params=pltpu.CompilerParams(use_tc_tiling_on_sc=False),
    scratch_types=[
        pltpu.VMEM((CHUNK, IW), jnp.int32),
        pltpu.VMEM((IW,), jnp.float32),
        pltpu.VMEM((2048,), jnp.float32),
        pltpu.VMEM_SHARED((N_PAD,), jnp.float32),
        pltpu.SemaphoreType.DMA,
    ],
)


@jax.jit
def kernel(edge_index, user_w, item_w, aspect_weight):
    edge_index = edge_index.astype(jnp.int32)
    src = edge_index[0]
    dst = edge_index[1] + N_USERS
    loops = jnp.arange(N, dtype=jnp.int32)
    row = jnp.concatenate([src, dst, loops])
    col = jnp.concatenate([dst, src, loops])
    pad = E_PAD - E
    pad_col = TRASH + (jnp.arange(pad, dtype=jnp.int32) % (N_PAD - N))
    row2d = jnp.concatenate([row, jnp.zeros((pad,), jnp.int32)]).reshape(E_ROWS, IW)
    col2d = jnp.concatenate([col, pad_col]).reshape(E_ROWS, IW)

    hist = _hist(col2d)
    deg = hist[0] + hist[1]
    deg = jnp.maximum(deg, 1.0)
    d2f = jnp.broadcast_to((1.0 / deg)[:, None], (N_PAD, 16)).reshape(1, -1)
    dinvf = jnp.broadcast_to(jax.lax.rsqrt(deg)[:, None], (N_PAD, 16)).reshape(1, -1)
    dsqf = 0.25 / dinvf

    x = jnp.concatenate([user_w, item_w], axis=0)
    x = jnp.pad(x, ((0, N_PAD - N), (0, 0)))
    x4f = jnp.transpose(x.reshape(N_PAD, 4, 16), (1, 0, 2)).reshape(4, -1)

    g = x4f * dinvf
    G = g
    for _ in range(N_LAYERS):
        S = _spmm(g.reshape(4, N_PAD, 16), row2d, col2d).reshape(4, -1)
        g = S * d2f
        G = G + g

    interest4 = (G * dsqf).reshape(4, N_PAD, 16)
    interest = jnp.transpose(interest4, (1, 0, 2)).reshape(N_PAD, D)[:N]

    alpha = jax.nn.softmax(aspect_weight, axis=0)
    user_final = alpha[0] * interest[:N_USERS] + alpha[1] * user_w
    item_final = interest[N_USERS:]
    return user_final, item_final


# speedup vs baseline: 1.8623x; 1.1584x over previous
import jax
import jax.numpy as jnp
from jax import lax
from jax.experimental import pallas as pl
from jax.experimental.pallas import tpu as pltpu
from jax.experimental.pallas import tpu_sc as plsc

N_USERS = 50000
N_ITEMS = 50000
N = N_USERS + N_ITEMS
D = 64
N_LAYERS = 3

N_PAD = 100352
TRASH = N
E = 2 * 500000 + N
E_PAD = 1105920
IW = 128
E_ROWS = E_PAD // IW
CHUNK = 6

NSUB = 16
ROWS_PER_SUB = N_PAD // NSUB
SUB_EROWS = E_ROWS // NSUB
SUB_CHUNKS = SUB_EROWS // CHUNK
NZCH = ROWS_PER_SUB // 128


def _zero_vmem(ref, nrows):
    zeros = jnp.zeros((16,), jnp.float32)

    def body(i, carry):
        ref[i, :] = zeros
        return carry

    lax.fori_loop(0, nrows, body, 0)


def _spmm_body(g4_hbm, row_hbm, col_hbm, s4_hbm, idxr_a, idxc_a, idxr_b,
               idxc_b, rows_a, rows_b, zb, sh, gsem_a, gsem_b, ssem_a,
               ssem_b):
    c = lax.axis_index("c")
    s = lax.axis_index("s")
    _zero_vmem(zb, 128)

    slots = ((idxr_a, idxc_a, rows_a, gsem_a, ssem_a),
             (idxr_b, idxc_b, rows_b, gsem_b, ssem_b))

    for blk_i in range(2):
        blk = 2 * c + blk_i
        g_blk = g4_hbm.at[blk]
        out_blk = s4_hbm.at[blk]

        def load_fire(chunk, slot):
            idxr, idxc, rows, gsem, _ = slots[slot]
            base = s * SUB_EROWS + chunk * CHUNK
            pltpu.sync_copy(row_hbm.at[pl.ds(base, CHUNK)], idxr)
            pltpu.sync_copy(col_hbm.at[pl.ds(base, CHUNK)], idxc)
            for j in range(CHUNK):
                pltpu.async_copy(g_blk.at[idxr.at[j]], rows.at[j], gsem)

        def wait_g(slot):
            idxr, _, rows, gsem, _ = slots[slot]
            for j in range(CHUNK):
                pltpu.make_async_copy(g_blk.at[idxr.at[j]], rows.at[j],
                                      gsem).wait()

        def fire_s(slot):
            _, idxc, rows, _, ssem = slots[slot]
            for j in range(CHUNK):
                pltpu.async_copy(rows.at[j], sh.at[idxc.at[j]], ssem,
                                 add=True)

        def wait_s(slot):
            _, idxc, rows, _, ssem = slots[slot]
            for j in range(CHUNK):
                pltpu.make_async_copy(rows.at[j], sh.at[idxc.at[j]],
                                      ssem).wait()

        def zero_body(i, carry):
            pltpu.sync_copy(zb, sh.at[pl.ds(s * ROWS_PER_SUB + i * 128, 128)])
            return carry

        lax.fori_loop(0, NZCH, zero_body, 0)
        plsc.subcore_barrier()

        def pair_body(i, carry):
            @pl.when(i > 0)
            def _():
                wait_s(0)
            load_fire(2 * i, 0)

            @pl.when(i > 0)
            def _():
                wait_g(1)
                fire_s(1)
                wait_s(1)
            load_fire(2 * i + 1, 1)
            wait_g(0)
            fire_s(0)
            return carry

        lax.fori_loop(0, SUB_CHUNKS // 2, pair_body, 0)
        wait_g(1)
        fire_s(1)
        wait_s(0)
        wait_s(1)
        plsc.subcore_barrier()

        pltpu.sync_copy(sh.at[pl.ds(s * ROWS_PER_SUB, ROWS_PER_SUB)],
                        out_blk.at[pl.ds(s * ROWS_PER_SUB, ROWS_PER_SUB)])
        plsc.subcore_barrier()


def _hist_body(col_hbm, out_hbm, idxc, ones_v, zb1, sh, sem):
    c = lax.axis_index("c")
    s = lax.axis_index("s")

    def zfill_body(i, carry):
        zb1[pl.ds(i * 16, 16)] = jnp.zeros((16,), jnp.float32)
        return carry

    lax.fori_loop(0, 2048 // 16, zfill_body, 0)

    def ones_body(i, carry):
        ones_v[pl.ds(i * 16, 16)] = jnp.ones((16,), jnp.float32)
        return carry

    lax.fori_loop(0, IW // 16, ones_body, 0)

    def zero_body(i, carry):
        pltpu.sync_copy(zb1, sh.at[pl.ds(s * ROWS_PER_SUB + i * 2048, 2048)])
        return carry

    nz = ROWS_PER_SUB // 2048
    lax.fori_loop(0, nz, zero_body, 0)
    rem = ROWS_PER_SUB - nz * 2048
    pltpu.sync_copy(zb1.at[pl.ds(0, rem)],
                    sh.at[pl.ds(s * ROWS_PER_SUB + nz * 2048, rem)])
    plsc.subcore_barrier()

    half_rows = E_ROWS // 2
    sub_rows = half_rows // NSUB
    nchunks = sub_rows // CHUNK

    def chunk_body(t, carry):
        base = c * half_rows + s * sub_rows + t * CHUNK
        pltpu.sync_copy(col_hbm.at[pl.ds(base, CHUNK)], idxc)
        adds = [
            pltpu.async_copy(ones_v, sh.at[idxc.at[j]], sem, add=True)
            for j in range(CHUNK)
        ]
        for d in adds:
            d.wait()
        return carry

    lax.fori_loop(0, nchunks, chunk_body, 0)
    plsc.subcore_barrier()

    pltpu.sync_copy(sh.at[pl.ds(s * ROWS_PER_SUB, ROWS_PER_SUB)],
                    out_hbm.at[c].at[pl.ds(s * ROWS_PER_SUB, ROWS_PER_SUB)])


_MESH = plsc.VectorSubcoreMesh(core_axis_name="c", subcore_axis_name="s")

_spmm = pl.kernel(
    _spmm_body,
    out_type=jax.ShapeDtypeStruct((4, N_PAD, 16), jnp.float32),
    mesh=_MESH,
    compiler_params=pltpu.CompilerParams(use_tc_tiling_on_sc=False),
    scratch_types=[
        pltpu.VMEM((CHUNK, IW), jnp.int32),
        pltpu.VMEM((CHUNK, IW), jnp.int32),
        pltpu.VMEM((CHUNK, IW), jnp.int32),
        pltpu.VMEM((CHUNK, IW), jnp.int32),
        pltpu.VMEM((CHUNK, IW, 16), jnp.float32),
        pltpu.VMEM((CHUNK, IW, 16), jnp.float32),
        pltpu.VMEM((128, 16), jnp.float32),
        pltpu.VMEM_SHARED((N_PAD, 16), jnp.float32),
        pltpu.SemaphoreType.DMA,
        pltpu.SemaphoreType.DMA,
        pltpu.SemaphoreType.DMA,
        pltpu.SemaphoreType.DMA,
    ],
)

_hist = pl.kernel(
    _hist_body,
    out_type=jax.ShapeDtypeStruct((2, N_PAD), jnp.float32),
    mesh=_MESH,
    compiler_params=pltpu.CompilerParams(use_tc_tiling_on_sc=False),
    scratch_types=[
        pltpu.VMEM((CHUNK, IW), jnp.int32),
        pltpu.VMEM((IW,), jnp.float32),
        pltpu.VMEM((2048,), jnp.float32),
        pltpu.VMEM_SHARED((N_PAD,), jnp.float32),
        pltpu.SemaphoreType.DMA,
    ],
)


@jax.jit
def kernel(edge_index, user_w, item_w, aspect_weight):
    edge_index = edge_index.astype(jnp.int32)
    src = edge_index[0]
    dst = edge_index[1] + N_USERS
    loops = jnp.arange(N, dtype=jnp.int32)
    row = jnp.concatenate([src, dst, loops])
    col = jnp.concatenate([dst, src, loops])
    pad = E_PAD - E
    pad_col = TRASH + (jnp.arange(pad, dtype=jnp.int32) % (N_PAD - N))
    row2d = jnp.concatenate([row, jnp.zeros((pad,), jnp.int32)]).reshape(E_ROWS, IW)
    col2d = jnp.concatenate([col, pad_col]).reshape(E_ROWS, IW)

    hist = _hist(col2d)
    deg = hist[0] + hist[1]
    deg = jnp.maximum(deg, 1.0)
    d2f = jnp.broadcast_to((1.0 / deg)[:, None], (N_PAD, 16)).reshape(1, -1)
    dinvf = jnp.broadcast_to(jax.lax.rsqrt(deg)[:, None], (N_PAD, 16)).reshape(1, -1)
    dsqf = 0.25 / dinvf

    x = jnp.concatenate([user_w, item_w], axis=0)
    x = jnp.pad(x, ((0, N_PAD - N), (0, 0)))
    x4f = jnp.transpose(x.reshape(N_PAD, 4, 16), (1, 0, 2)).reshape(4, -1)

    g = x4f * dinvf
    G = g
    for _ in range(N_LAYERS):
        S = _spmm(g.reshape(4, N_PAD, 16), row2d, col2d).reshape(4, -1)
        g = S * d2f
        G = G + g

    interest4 = (G * dsqf).reshape(4, N_PAD, 16)
    interest = jnp.transpose(interest4, (1, 0, 2)).reshape(N_PAD, D)[:N]

    alpha = jax.nn.softmax(aspect_weight, axis=0)
    user_final = alpha[0] * interest[:N_USERS] + alpha[1] * user_w
    item_final = interest[N_USERS:]
    return user_final, item_final
